# baseline (speedup 1.0000x reference)
import jax
import jax.numpy as jnp
from jax import lax
from jax.experimental import pallas as pl

ROWS, COLS = 128, 32768

_RBLK = 32
_W = 128
_NCHUNK = COLS // _W


def _body(x_ref, y_ref, amax_ref):
    blk = x_ref[...]
    y_ref[...] = blk
    acc = blk[:, :_W]
    idx = jnp.zeros((_RBLK, _W), dtype=jnp.int32)
    for c in range(1, _NCHUNK):
        v = blk[:, c * _W:(c + 1) * _W]
        m = v > acc
        acc = jnp.where(m, v, acc)
        idx = jnp.where(m, c, idx)
    bmax = jnp.max(acc, axis=1, keepdims=True)
    gidx = idx * _W + lax.broadcasted_iota(jnp.int32, (_RBLK, _W), 1)
    cand = jnp.where(acc == bmax, gidx, COLS)
    amax_ref[...] = jnp.min(cand, axis=1, keepdims=True)


def kernel(inputs):
    y, _idx = pl.pallas_call(
        _body,
        grid=(ROWS // _RBLK,),
        in_specs=[pl.BlockSpec((_RBLK, COLS), lambda k: (k, 0))],
        out_specs=[
            pl.BlockSpec((_RBLK, COLS), lambda k: (k, 0)),
            pl.BlockSpec((_RBLK, 1), lambda k: (k, 0)),
        ],
        out_shape=[
            jax.ShapeDtypeStruct((ROWS, COLS), jnp.float32),
            jax.ShapeDtypeStruct((ROWS, 1), jnp.int32),
        ],
    )(inputs)
    return y


# speedup vs baseline: 1.1723x; 1.1723x over previous
import jax
import jax.numpy as jnp
from jax import lax
from jax.experimental import pallas as pl
from jax.experimental.pallas import tpu as pltpu

ROWS, COLS = 128, 32768

_CH = 8
_NCH = ROWS // _CH
_W = 128
_NCHUNK = COLS // _W


def _chunk_argmax(blk):
    acc = blk[:, :_W]
    idx = jnp.zeros((_CH, _W), dtype=jnp.int32)
    for c in range(1, _NCHUNK):
        v = blk[:, c * _W:(c + 1) * _W]
        m = v > acc
        acc = jnp.where(m, v, acc)
        idx = jnp.where(m, c, idx)
    bmax = jnp.max(acc, axis=1, keepdims=True)
    gidx = idx * _W + lax.broadcasted_iota(jnp.int32, (_CH, _W), 1)
    cand = jnp.where(acc == bmax, gidx, COLS)
    return jnp.min(cand, axis=1, keepdims=True)


def _body(x_hbm, y_hbm, amax_ref, buf, rsem, wsem):
    reads = [
        pltpu.make_async_copy(
            x_hbm.at[pl.ds(i * _CH, _CH)], buf.at[i], rsem.at[i]
        )
        for i in range(_NCH)
    ]
    writes = [
        pltpu.make_async_copy(
            buf.at[i], y_hbm.at[pl.ds(i * _CH, _CH)], wsem.at[i]
        )
        for i in range(_NCH)
    ]
    for c in reads:
        c.start()
    for i in range(_NCH):
        reads[i].wait()
        writes[i].start()
        amax_ref[pl.ds(i * _CH, _CH), :] = _chunk_argmax(buf[i])
    for c in writes:
        c.wait()


def kernel(inputs):
    y, _idx = pl.pallas_call(
        _body,
        in_specs=[pl.BlockSpec(memory_space=pl.ANY)],
        out_specs=[
            pl.BlockSpec(memory_space=pl.ANY),
            pl.BlockSpec((ROWS, 1), lambda: (0, 0)),
        ],
        out_shape=[
            jax.ShapeDtypeStruct((ROWS, COLS), jnp.float32),
            jax.ShapeDtypeStruct((ROWS, 1), jnp.int32),
        ],
        scratch_shapes=[
            pltpu.VMEM((_NCH, _CH, COLS), jnp.float32),
            pltpu.SemaphoreType.DMA((_NCH,)),
            pltpu.SemaphoreType.DMA((_NCH,)),
        ],
    )(inputs)
    return y
